# baseline (speedup 1.0000x reference)
import functools

import jax
import jax.numpy as jnp
from jax import lax
from jax.experimental import pallas as pl
from jax.experimental.pallas import tpu as pltpu
from jax.experimental.pallas import tpu_sc as plsc

NUM_CONCEPTS = 1000000
D = 64
B = 16384

RC = 8192
NSTEP = 62
PACKED_ROWS = NSTEP * RC

NC, NS = 2, 16
NW = NC * NS
BPW = B // NW
CHUNK = 128


def _repack_body(x_ref, out_ref):
    x = x_ref[...]
    xb = x.astype(jnp.bfloat16)
    eye = jnp.eye(D, dtype=jnp.bfloat16)
    left = lax.dot_general(xb[:, :RC], eye, (((0,), (0,)), ((), ())),
                           preferred_element_type=jnp.float32)
    right = lax.dot_general(xb[:, RC:], eye, (((0,), (0,)), ((), ())),
                            preferred_element_type=jnp.float32)
    out_ref[...] = jnp.concatenate([left, right], axis=1)


def _repack(table_t):
    return pl.pallas_call(
        _repack_body,
        grid=(NSTEP,),
        in_specs=[pl.BlockSpec((D, 2 * RC), lambda i: (0, i))],
        out_specs=pl.BlockSpec((RC, 2 * D), lambda i: (i, 0)),
        out_shape=jax.ShapeDtypeStruct((PACKED_ROWS, 2 * D), jnp.float32),
    )(table_t)


_sc_mesh = plsc.VectorSubcoreMesh(core_axis_name="c", subcore_axis_name="s")


@functools.partial(
    pl.kernel,
    out_type=(
        jax.ShapeDtypeStruct((B, 2 * D), jnp.float32),
        jax.ShapeDtypeStruct((B, 2 * D), jnp.float32),
    ),
    mesh=_sc_mesh,
    compiler_params=pltpu.CompilerParams(use_tc_tiling_on_sc=True),
    scratch_types=[
        pltpu.VMEM((BPW,), jnp.int32),
        pltpu.VMEM((BPW,), jnp.int32),
        pltpu.VMEM((2, CHUNK, 2 * D), jnp.float32),
        pltpu.VMEM((2, CHUNK, 2 * D), jnp.float32),
        pltpu.SemaphoreType.DMA,
        pltpu.SemaphoreType.DMA,
    ],
)
def _sc_gather(table_hbm, idx1_hbm, idx2_hbm, out1_hbm, out2_hbm,
               idx1_v, idx2_v, rows1_v, rows2_v, gsem, wsem):
    wid = lax.axis_index("s") * NC + lax.axis_index("c")
    base = wid * BPW
    pltpu.sync_copy(idx1_hbm.at[pl.ds(base, BPW)], idx1_v)
    pltpu.sync_copy(idx2_hbm.at[pl.ds(base, BPW)], idx2_v)
    for j in range(BPW // 16):
        s = pl.ds(j * 16, 16)
        v1 = idx1_v[s]
        idx1_v[s] = ((v1 >> 14) << 13) | (v1 & (RC - 1))
        v2 = idx2_v[s]
        idx2_v[s] = ((v2 >> 14) << 13) | (v2 & (RC - 1))
    writes = [None, None]
    for r in range(BPW // CHUNK):
        a = r % 2
        if writes[a] is not None:
            for w in writes[a]:
                w.wait()
        g1 = pltpu.async_copy(
            table_hbm.at[idx1_v.at[pl.ds(r * CHUNK, CHUNK)]],
            rows1_v.at[a], gsem)
        g2 = pltpu.async_copy(
            table_hbm.at[idx2_v.at[pl.ds(r * CHUNK, CHUNK)]],
            rows2_v.at[a], gsem)
        g1.wait()
        g2.wait()
        writes[a] = [
            pltpu.async_copy(rows1_v.at[a],
                             out1_hbm.at[pl.ds(base + r * CHUNK, CHUNK)],
                             wsem),
            pltpu.async_copy(rows2_v.at[a],
                             out2_hbm.at[pl.ds(base + r * CHUNK, CHUNK)],
                             wsem),
        ]
    for ws in writes:
        for w in ws:
            w.wait()


BLK = 2048


def _mlp_body(g1_ref, g2_ref, p1_ref, p2_ref, w1a_ref, w1b_ref, b1_ref,
              w2_ref, b2_ref, w3_ref, b3_ref, out_ref):
    g1 = g1_ref[...]
    g2 = g2_ref[...]
    p1 = p1_ref[...]
    p2 = p2_ref[...]
    e1 = jnp.where(p1 > 0.5, g1[:, D:], g1[:, :D])
    e2 = jnp.where(p2 > 0.5, g2[:, D:], g2[:, :D])
    h = jnp.dot(e1, w1a_ref[...], preferred_element_type=jnp.float32)
    h = h + jnp.dot(e2, w1b_ref[...], preferred_element_type=jnp.float32)
    h = jnp.maximum(h + b1_ref[...], 0.0)
    h2 = jnp.dot(h, w2_ref[...], preferred_element_type=jnp.float32)
    h2 = jnp.maximum(h2 + b2_ref[...], 0.0)
    logit = jnp.sum(h2 * w3_ref[...], axis=1, keepdims=True) + b3_ref[...]
    out_ref[...] = jax.nn.sigmoid(logit)


def _mlp(g1, g2, p1, p2, w1a, w1b, b1, w2, b2, w3_row, b3):
    grid = (B // BLK,)
    return pl.pallas_call(
        _mlp_body,
        grid=grid,
        in_specs=[
            pl.BlockSpec((BLK, 2 * D), lambda i: (i, 0)),
            pl.BlockSpec((BLK, 2 * D), lambda i: (i, 0)),
            pl.BlockSpec((BLK, 1), lambda i: (i, 0)),
            pl.BlockSpec((BLK, 1), lambda i: (i, 0)),
            pl.BlockSpec((D, 256), lambda i: (0, 0)),
            pl.BlockSpec((D, 256), lambda i: (0, 0)),
            pl.BlockSpec((1, 256), lambda i: (0, 0)),
            pl.BlockSpec((256, 128), lambda i: (0, 0)),
            pl.BlockSpec((1, 128), lambda i: (0, 0)),
            pl.BlockSpec((1, 128), lambda i: (0, 0)),
            pl.BlockSpec((1, 1), lambda i: (0, 0)),
        ],
        out_specs=pl.BlockSpec((BLK, 1), lambda i: (i, 0)),
        out_shape=jax.ShapeDtypeStruct((B, 1), jnp.float32),
    )(g1, g2, p1, p2, w1a, w1b, b1, w2, b2, w3_row, b3)


def kernel(concept_table, W1, b1, W2, b2, W3, b3, concept1_idx, concept2_idx):
    table2 = _repack(concept_table.T)
    i1 = concept1_idx.astype(jnp.int32)
    i2 = concept2_idx.astype(jnp.int32)
    g1, g2 = _sc_gather(table2, i1, i2)
    p1 = ((i1 >> 13) & 1).astype(jnp.float32).reshape(B, 1)
    p2 = ((i2 >> 13) & 1).astype(jnp.float32).reshape(B, 1)
    w1a = W1[:D]
    w1b = W1[D:]
    return _mlp(g1, g2, p1, p2, w1a, w1b,
                b1.reshape(1, 256), W2, b2.reshape(1, 128),
                W3.reshape(1, 128), b3.reshape(1, 1))


# speedup vs baseline: 1.0830x; 1.0830x over previous
import functools

import jax
import jax.numpy as jnp
from jax import lax
from jax.experimental import pallas as pl
from jax.experimental.pallas import tpu as pltpu
from jax.experimental.pallas import tpu_sc as plsc

NUM_CONCEPTS = 1000000
D = 64
B = 16384

RC = 16384
LOG_RC = 14
NSTEP = 31
PACKED_ROWS = NSTEP * RC

NC, NS = 2, 16
NW = NC * NS
BPW = B // NW
CHUNK = 128


def _repack_body(x_ref, out_ref):
    x = x_ref[...]
    xb = x.astype(jnp.bfloat16)
    eye = jnp.eye(D, dtype=jnp.bfloat16)
    left = lax.dot_general(xb[:, :RC], eye, (((0,), (0,)), ((), ())),
                           preferred_element_type=jnp.float32)
    right = lax.dot_general(xb[:, RC:], eye, (((0,), (0,)), ((), ())),
                            preferred_element_type=jnp.float32)
    out_ref[...] = jnp.concatenate([left, right], axis=1)


def _repack(table_t):
    return pl.pallas_call(
        _repack_body,
        grid=(NSTEP,),
        in_specs=[pl.BlockSpec((D, 2 * RC), lambda i: (0, i))],
        out_specs=pl.BlockSpec((RC, 2 * D), lambda i: (i, 0)),
        out_shape=jax.ShapeDtypeStruct((PACKED_ROWS, 2 * D), jnp.float32),
    )(table_t)


_sc_mesh = plsc.VectorSubcoreMesh(core_axis_name="c", subcore_axis_name="s")


@functools.partial(
    pl.kernel,
    out_type=(
        jax.ShapeDtypeStruct((B, 2 * D), jnp.float32),
        jax.ShapeDtypeStruct((B, 2 * D), jnp.float32),
    ),
    mesh=_sc_mesh,
    compiler_params=pltpu.CompilerParams(use_tc_tiling_on_sc=True),
    scratch_types=[
        pltpu.VMEM((BPW,), jnp.int32),
        pltpu.VMEM((BPW,), jnp.int32),
        pltpu.VMEM((2, CHUNK, 2 * D), jnp.float32),
        pltpu.VMEM((2, CHUNK, 2 * D), jnp.float32),
        pltpu.SemaphoreType.DMA,
        pltpu.SemaphoreType.DMA,
    ],
)
def _sc_gather(table_hbm, idx1_hbm, idx2_hbm, out1_hbm, out2_hbm,
               idx1_v, idx2_v, rows1_v, rows2_v, gsem, wsem):
    wid = lax.axis_index("s") * NC + lax.axis_index("c")
    base = wid * BPW
    pltpu.sync_copy(idx1_hbm.at[pl.ds(base, BPW)], idx1_v)
    pltpu.sync_copy(idx2_hbm.at[pl.ds(base, BPW)], idx2_v)
    for j in range(BPW // 16):
        s = pl.ds(j * 16, 16)
        v1 = idx1_v[s]
        idx1_v[s] = ((v1 >> (LOG_RC + 1)) << LOG_RC) | (v1 & (RC - 1))
        v2 = idx2_v[s]
        idx2_v[s] = ((v2 >> (LOG_RC + 1)) << LOG_RC) | (v2 & (RC - 1))
    writes = [None, None]
    for r in range(BPW // CHUNK):
        a = r % 2
        if writes[a] is not None:
            for w in writes[a]:
                w.wait()
        g1 = pltpu.async_copy(
            table_hbm.at[idx1_v.at[pl.ds(r * CHUNK, CHUNK)]],
            rows1_v.at[a], gsem)
        g2 = pltpu.async_copy(
            table_hbm.at[idx2_v.at[pl.ds(r * CHUNK, CHUNK)]],
            rows2_v.at[a], gsem)
        g1.wait()
        g2.wait()
        writes[a] = [
            pltpu.async_copy(rows1_v.at[a],
                             out1_hbm.at[pl.ds(base + r * CHUNK, CHUNK)],
                             wsem),
            pltpu.async_copy(rows2_v.at[a],
                             out2_hbm.at[pl.ds(base + r * CHUNK, CHUNK)],
                             wsem),
        ]
    for ws in writes:
        for w in ws:
            w.wait()


BLK = 2048


def _mlp_body(g1_ref, g2_ref, p1_ref, p2_ref, w1a_ref, w1b_ref, b1_ref,
              w2_ref, b2_ref, w3_ref, b3_ref, out_ref):
    g1 = g1_ref[...]
    g2 = g2_ref[...]
    p1 = p1_ref[...]
    p2 = p2_ref[...]
    e1 = jnp.where(p1 > 0.5, g1[:, D:], g1[:, :D])
    e2 = jnp.where(p2 > 0.5, g2[:, D:], g2[:, :D])
    h = jnp.dot(e1, w1a_ref[...], preferred_element_type=jnp.float32)
    h = h + jnp.dot(e2, w1b_ref[...], preferred_element_type=jnp.float32)
    h = jnp.maximum(h + b1_ref[...], 0.0)
    h2 = jnp.dot(h, w2_ref[...], preferred_element_type=jnp.float32)
    h2 = jnp.maximum(h2 + b2_ref[...], 0.0)
    logit = jnp.sum(h2 * w3_ref[...], axis=1, keepdims=True) + b3_ref[...]
    out_ref[...] = jax.nn.sigmoid(logit)


def _mlp(g1, g2, p1, p2, w1a, w1b, b1, w2, b2, w3_row, b3):
    grid = (B // BLK,)
    return pl.pallas_call(
        _mlp_body,
        grid=grid,
        in_specs=[
            pl.BlockSpec((BLK, 2 * D), lambda i: (i, 0)),
            pl.BlockSpec((BLK, 2 * D), lambda i: (i, 0)),
            pl.BlockSpec((BLK, 1), lambda i: (i, 0)),
            pl.BlockSpec((BLK, 1), lambda i: (i, 0)),
            pl.BlockSpec((D, 256), lambda i: (0, 0)),
            pl.BlockSpec((D, 256), lambda i: (0, 0)),
            pl.BlockSpec((1, 256), lambda i: (0, 0)),
            pl.BlockSpec((256, 128), lambda i: (0, 0)),
            pl.BlockSpec((1, 128), lambda i: (0, 0)),
            pl.BlockSpec((1, 128), lambda i: (0, 0)),
            pl.BlockSpec((1, 1), lambda i: (0, 0)),
        ],
        out_specs=pl.BlockSpec((BLK, 1), lambda i: (i, 0)),
        out_shape=jax.ShapeDtypeStruct((B, 1), jnp.float32),
    )(g1, g2, p1, p2, w1a, w1b, b1, w2, b2, w3_row, b3)


def kernel(concept_table, W1, b1, W2, b2, W3, b3, concept1_idx, concept2_idx):
    table2 = _repack(concept_table.T)
    i1 = concept1_idx.astype(jnp.int32)
    i2 = concept2_idx.astype(jnp.int32)
    g1, g2 = _sc_gather(table2, i1, i2)
    p1 = ((i1 >> LOG_RC) & 1).astype(jnp.float32).reshape(B, 1)
    p2 = ((i2 >> LOG_RC) & 1).astype(jnp.float32).reshape(B, 1)
    w1a = W1[:D]
    w1b = W1[D:]
    return _mlp(g1, g2, p1, p2, w1a, w1b,
                b1.reshape(1, 256), W2, b2.reshape(1, 128),
                W3.reshape(1, 128), b3.reshape(1, 1))


# speedup vs baseline: 1.0870x; 1.0037x over previous
import functools

import jax
import jax.numpy as jnp
from jax import lax
from jax.experimental import pallas as pl
from jax.experimental.pallas import tpu as pltpu
from jax.experimental.pallas import tpu_sc as plsc

NUM_CONCEPTS = 1000000
D = 64
B = 16384

RC = 16384
LOG_RC = 14
NSTEP = 31
PACKED_ROWS = NSTEP * RC

NC, NS = 2, 16
NW = NC * NS
BPW = B // NW
CHUNK = 128


def _repack_body(x_ref, out_ref):
    x = x_ref[...]
    xb = x.astype(jnp.bfloat16)
    eye = jnp.eye(D, dtype=jnp.bfloat16)
    left = lax.dot_general(xb[:, :RC], eye, (((0,), (0,)), ((), ())),
                           preferred_element_type=jnp.float32)
    right = lax.dot_general(xb[:, RC:], eye, (((0,), (0,)), ((), ())),
                            preferred_element_type=jnp.float32)
    out_ref[...] = jnp.concatenate([left, right], axis=1)


def _repack(table_t):
    return pl.pallas_call(
        _repack_body,
        grid=(NSTEP,),
        in_specs=[pl.BlockSpec((D, 2 * RC), lambda i: (0, i))],
        out_specs=pl.BlockSpec((RC, 2 * D), lambda i: (i, 0)),
        out_shape=jax.ShapeDtypeStruct((PACKED_ROWS, 2 * D), jnp.float32),
    )(table_t)


_sc_mesh = plsc.VectorSubcoreMesh(core_axis_name="c", subcore_axis_name="s")


@functools.partial(
    pl.kernel,
    out_type=(
        jax.ShapeDtypeStruct((B, 2 * D), jnp.float32),
        jax.ShapeDtypeStruct((B, 2 * D), jnp.float32),
    ),
    mesh=_sc_mesh,
    compiler_params=pltpu.CompilerParams(use_tc_tiling_on_sc=True),
    scratch_types=[
        pltpu.VMEM((BPW,), jnp.int32),
        pltpu.VMEM((BPW,), jnp.int32),
        pltpu.VMEM((2, CHUNK, 2 * D), jnp.float32),
        pltpu.VMEM((2, CHUNK, 2 * D), jnp.float32),
        pltpu.SemaphoreType.DMA,
        pltpu.SemaphoreType.DMA,
    ],
)
def _sc_gather(table_hbm, idx1_hbm, idx2_hbm, out1_hbm, out2_hbm,
               idx1_v, idx2_v, rows1_v, rows2_v, gsem, wsem):
    wid = lax.axis_index("s") * NC + lax.axis_index("c")
    base = wid * BPW
    pltpu.sync_copy(idx1_hbm.at[pl.ds(base, BPW)], idx1_v)
    pltpu.sync_copy(idx2_hbm.at[pl.ds(base, BPW)], idx2_v)
    for j in range(BPW // 16):
        s = pl.ds(j * 16, 16)
        v1 = idx1_v[s]
        idx1_v[s] = ((v1 >> (LOG_RC + 1)) << LOG_RC) | (v1 & (RC - 1))
        v2 = idx2_v[s]
        idx2_v[s] = ((v2 >> (LOG_RC + 1)) << LOG_RC) | (v2 & (RC - 1))
    writes = [None, None]
    for r in range(BPW // CHUNK):
        a = r % 2
        if writes[a] is not None:
            for w in writes[a]:
                w.wait()
        g1 = pltpu.async_copy(
            table_hbm.at[idx1_v.at[pl.ds(r * CHUNK, CHUNK)]],
            rows1_v.at[a], gsem)
        g2 = pltpu.async_copy(
            table_hbm.at[idx2_v.at[pl.ds(r * CHUNK, CHUNK)]],
            rows2_v.at[a], gsem)
        g1.wait()
        g2.wait()
        writes[a] = [
            pltpu.async_copy(rows1_v.at[a],
                             out1_hbm.at[pl.ds(base + r * CHUNK, CHUNK)],
                             wsem),
            pltpu.async_copy(rows2_v.at[a],
                             out2_hbm.at[pl.ds(base + r * CHUNK, CHUNK)],
                             wsem),
        ]
    for ws in writes:
        for w in ws:
            w.wait()


BLK = 4096


def _mlp_body(g1_ref, g2_ref, p1_ref, p2_ref, w1a_ref, w1b_ref, b1_ref,
              w2_ref, b2_ref, w3_ref, b3_ref, out_ref):
    g1 = g1_ref[...]
    g2 = g2_ref[...]
    p1 = p1_ref[...]
    p2 = p2_ref[...]
    e1 = jnp.where(p1 > 0.5, g1[:, D:], g1[:, :D])
    e2 = jnp.where(p2 > 0.5, g2[:, D:], g2[:, :D])
    h = jnp.dot(e1, w1a_ref[...], preferred_element_type=jnp.float32)
    h = h + jnp.dot(e2, w1b_ref[...], preferred_element_type=jnp.float32)
    h = jnp.maximum(h + b1_ref[...], 0.0)
    h2 = jnp.dot(h, w2_ref[...], preferred_element_type=jnp.float32)
    h2 = jnp.maximum(h2 + b2_ref[...], 0.0)
    logit = jnp.sum(h2 * w3_ref[...], axis=1, keepdims=True) + b3_ref[...]
    out_ref[...] = jax.nn.sigmoid(logit)


def _mlp(g1, g2, p1, p2, w1a, w1b, b1, w2, b2, w3_row, b3):
    grid = (B // BLK,)
    return pl.pallas_call(
        _mlp_body,
        grid=grid,
        in_specs=[
            pl.BlockSpec((BLK, 2 * D), lambda i: (i, 0)),
            pl.BlockSpec((BLK, 2 * D), lambda i: (i, 0)),
            pl.BlockSpec((BLK, 1), lambda i: (i, 0)),
            pl.BlockSpec((BLK, 1), lambda i: (i, 0)),
            pl.BlockSpec((D, 256), lambda i: (0, 0)),
            pl.BlockSpec((D, 256), lambda i: (0, 0)),
            pl.BlockSpec((1, 256), lambda i: (0, 0)),
            pl.BlockSpec((256, 128), lambda i: (0, 0)),
            pl.BlockSpec((1, 128), lambda i: (0, 0)),
            pl.BlockSpec((1, 128), lambda i: (0, 0)),
            pl.BlockSpec((1, 1), lambda i: (0, 0)),
        ],
        out_specs=pl.BlockSpec((BLK, 1), lambda i: (i, 0)),
        out_shape=jax.ShapeDtypeStruct((B, 1), jnp.float32),
    )(g1, g2, p1, p2, w1a, w1b, b1, w2, b2, w3_row, b3)


def kernel(concept_table, W1, b1, W2, b2, W3, b3, concept1_idx, concept2_idx):
    table2 = _repack(concept_table.T)
    i1 = concept1_idx.astype(jnp.int32)
    i2 = concept2_idx.astype(jnp.int32)
    g1, g2 = _sc_gather(table2, i1, i2)
    p1 = ((i1 >> LOG_RC) & 1).astype(jnp.float32).reshape(B, 1)
    p2 = ((i2 >> LOG_RC) & 1).astype(jnp.float32).reshape(B, 1)
    w1a = W1[:D]
    w1b = W1[D:]
    return _mlp(g1, g2, p1, p2, w1a, w1b,
                b1.reshape(1, 256), W2, b2.reshape(1, 128),
                W3.reshape(1, 128), b3.reshape(1, 1))
